# baseline (speedup 1.0000x reference)
import functools

import jax
import jax.numpy as jnp
from jax import lax
from jax.experimental import pallas as pl
from jax.experimental.pallas import tpu as pltpu
from jax.experimental.pallas import tpu_sc as plsc

NFEAT = 1000000
NEMB = 16
B = 16384
F = 26
NW = 32
RW = B // NW
RC = 64
NCHUNK = RW // RC
CF = RC * F

_mesh = plsc.VectorSubcoreMesh(core_axis_name="c", subcore_axis_name="s")

TCOL = 31248
TCB = 3472
TNCH = TCOL // TCB
TTAIL = NFEAT - NW * TCOL


@functools.partial(
    pl.kernel,
    out_type=jax.ShapeDtypeStruct((NFEAT, NEMB), jnp.float32),
    mesh=_mesh,
    compiler_params=pltpu.CompilerParams(use_tc_tiling_on_sc=False, needs_layout_passes=False),
    scratch_types=[
        pltpu.VMEM((NEMB, TCB), jnp.float32),
        pltpu.VMEM((TCB, NEMB), jnp.float32),
    ],
)
def _table_transpose(tab_hbm, out_hbm, in_v, out_v):
    wid = lax.axis_index("s") * 2 + lax.axis_index("c")
    start = wid * TCOL
    lanes = lax.iota(jnp.int32, 16)

    def chunk_body(k, carry):
        c0 = start + k * TCB
        pltpu.sync_copy(tab_hbm.at[:, pl.ds(c0, TCB)], in_v)

        def col_body(i, carry2):
            out_v[i, :] = plsc.load_gather(in_v, [lanes, jnp.full((16,), i, jnp.int32)])
            return carry2

        lax.fori_loop(0, TCB, col_body, 0)
        pltpu.sync_copy(out_v, out_hbm.at[pl.ds(c0, TCB), :])
        return carry

    lax.fori_loop(0, TNCH, chunk_body, 0)

    @pl.when(wid == NW - 1)
    def _tail():
        t0 = NW * TCOL
        pltpu.sync_copy(tab_hbm.at[:, pl.ds(t0, TTAIL)], in_v.at[:, pl.ds(0, TTAIL)])

        def tcol_body(i, carry2):
            out_v[i, :] = plsc.load_gather(in_v, [lanes, jnp.full((16,), i, jnp.int32)])
            return carry2

        lax.fori_loop(0, TTAIL, tcol_body, 0)
        pltpu.sync_copy(out_v.at[pl.ds(0, TTAIL), :], out_hbm.at[pl.ds(t0, TTAIL), :])


@functools.partial(
    pl.kernel,
    out_type=jax.ShapeDtypeStruct((F, NEMB, B), jnp.float32),
    mesh=_mesh,
    compiler_params=pltpu.CompilerParams(use_tc_tiling_on_sc=False, needs_layout_passes=False),
    scratch_types=[
        pltpu.VMEM((F, RW), jnp.int32),
        pltpu.VMEM((F, RW), jnp.float32),
        pltpu.VMEM((RW * F,), jnp.int32),
        pltpu.VMEM((CF, NEMB), jnp.float32),
        pltpu.VMEM((F, NEMB, RC), jnp.float32),
        pltpu.SemaphoreType.DMA,
    ],
)
def _emb_lookup(ids_hbm, vals_hbm, table_hbm, out_hbm,
                ids_v, vals_v, idx_v, rows_v, outc_v, sem):
    wid = lax.axis_index("s") * 2 + lax.axis_index("c")
    b0 = wid * RW
    pltpu.sync_copy(ids_hbm.at[:, pl.ds(b0, RW)], ids_v)
    pltpu.sync_copy(vals_hbm.at[:, pl.ds(b0, RW)], vals_v)

    lanes = lax.iota(jnp.int32, 16)
    lanes_hi = lanes + (F - 16)

    def flat_body(i, carry):
        a0 = plsc.load_gather(ids_v, [lanes, jnp.full((16,), i, jnp.int32)])
        a1 = plsc.load_gather(ids_v, [lanes_hi, jnp.full((16,), i, jnp.int32)])
        plsc.store_scatter(idx_v, [i * F + lanes], a0)
        plsc.store_scatter(idx_v, [i * F + (F - 16) + lanes], a1)
        return carry

    lax.fori_loop(0, RW, flat_body, 0)

    def chunk_body(k, carry):
        off = pl.multiple_of(k * CF, 8)
        pltpu.async_copy(table_hbm.at[idx_v.at[pl.ds(off, CF)]], rows_v, sem).wait()

        def row_body(i, carry2):
            bcol = jnp.full((16,), i, jnp.int32)
            vv0 = plsc.load_gather(vals_v, [lanes, k * RC + bcol])
            vv1 = plsc.load_gather(vals_v, [lanes_hi, k * RC + bcol])
            for j in range(F):
                v = vv0[j] if j < 16 else vv1[j - (F - 16)]
                plsc.store_scatter(
                    outc_v,
                    [jnp.full((16,), j, jnp.int32), lanes, bcol],
                    rows_v[i * F + j, :] * v,
                )
            return carry2

        lax.fori_loop(0, RC, row_body, 0)
        pltpu.sync_copy(outc_v, out_hbm.at[:, :, pl.ds(b0 + k * RC, RC)])
        return carry

    lax.fori_loop(0, NCHUNK, chunk_body, 0)


def kernel(ids, vals, table):
    table_rm = _table_transpose(table.T)
    out_t = _emb_lookup(ids.astype(jnp.int32).T, vals.T, table_rm)
    return jnp.transpose(out_t, (2, 0, 1))


# speedup vs baseline: 3.9669x; 3.9669x over previous
import functools

import jax
import jax.numpy as jnp
from jax import lax
from jax.experimental import pallas as pl
from jax.experimental.pallas import tpu as pltpu
from jax.experimental.pallas import tpu_sc as plsc

NFEAT = 1000000
NEMB = 16
B = 16384
F = 26
NW = 32
RW = B // NW
RC = 64
NCHUNK = RW // RC
CF = RC * F

_mesh = plsc.VectorSubcoreMesh(core_axis_name="c", subcore_axis_name="s")

TROWS = 1000064
TFLAT = TROWS * NEMB // 128
TCOL = 31232
TCB = 512
TNCH = TCOL // TCB
TTAIL = NFEAT - NW * TCOL


@functools.partial(
    pl.kernel,
    out_type=jax.ShapeDtypeStruct((TFLAT, 128), jnp.float32),
    mesh=_mesh,
    compiler_params=pltpu.CompilerParams(use_tc_tiling_on_sc=True, needs_layout_passes=False),
    scratch_types=[
        pltpu.VMEM((NEMB, TCB), jnp.float32),
        pltpu.VMEM((TCB * NEMB // 128, 128), jnp.float32),
    ],
)
def _table_transpose(tab_hbm, tail_hbm, out_hbm, in_v, out_v):
    wid = lax.axis_index("s") * 2 + lax.axis_index("c")
    start = wid * TCOL
    lanes = lax.iota(jnp.int32, 16)

    def do_chunk(c0, ncols):
        c0 = pl.multiple_of(c0, 64)
        pltpu.sync_copy(tab_hbm.at[:, pl.ds(c0, ncols)],
                        in_v.at[:, pl.ds(0, ncols)])

        def grp_body(g, carry2):
            cg = g * 16
            rows16 = (cg + lanes) >> 3
            colbase = ((cg + lanes) & 7) * 16
            for e in range(NEMB):
                plsc.store_scatter(out_v, [rows16, colbase + e],
                                   in_v[e, pl.ds(cg, 16)])
            return carry2

        lax.fori_loop(0, ncols // 16, grp_body, 0)
        r0 = pl.multiple_of(c0 * NEMB // 128, 8)
        pltpu.sync_copy(out_v.at[pl.ds(0, ncols * NEMB // 128), :],
                        out_hbm.at[pl.ds(r0, ncols * NEMB // 128), :])

    def chunk_body(k, carry):
        do_chunk(start + k * TCB, TCB)
        return carry

    lax.fori_loop(0, TNCH, chunk_body, 0)

    @pl.when(wid == NW - 1)
    def _tail():
        do_chunk(NW * TCOL, TCB)
        pltpu.sync_copy(tail_hbm, out_hbm.at[pl.ds((NFEAT - 64) * NEMB // 128, 8), :])


@functools.partial(
    pl.kernel,
    out_type=jax.ShapeDtypeStruct((F, NEMB, B), jnp.float32),
    mesh=_mesh,
    compiler_params=pltpu.CompilerParams(use_tc_tiling_on_sc=False, needs_layout_passes=False),
    scratch_types=[
        pltpu.VMEM((F, RW), jnp.int32),
        pltpu.VMEM((F, RW), jnp.float32),
        pltpu.VMEM((RW * F,), jnp.int32),
        pltpu.VMEM((CF, NEMB), jnp.float32),
        pltpu.VMEM((F, NEMB, RC), jnp.float32),
        pltpu.SemaphoreType.DMA,
    ],
)
def _emb_lookup(ids_hbm, vals_hbm, table_hbm, out_hbm,
                ids_v, vals_v, idx_v, rows_v, outc_v, sem):
    wid = lax.axis_index("s") * 2 + lax.axis_index("c")
    b0 = wid * RW
    pltpu.sync_copy(ids_hbm.at[:, pl.ds(b0, RW)], ids_v)
    pltpu.sync_copy(vals_hbm.at[:, pl.ds(b0, RW)], vals_v)

    lanes = lax.iota(jnp.int32, 16)
    lanes_hi = lanes + (F - 16)

    def flat_body(i, carry):
        a0 = plsc.load_gather(ids_v, [lanes, jnp.full((16,), i, jnp.int32)])
        a1 = plsc.load_gather(ids_v, [lanes_hi, jnp.full((16,), i, jnp.int32)])
        plsc.store_scatter(idx_v, [i * F + lanes], a0)
        plsc.store_scatter(idx_v, [i * F + (F - 16) + lanes], a1)
        return carry

    lax.fori_loop(0, RW, flat_body, 0)

    def chunk_body(k, carry):
        off = pl.multiple_of(k * CF, 8)
        pltpu.async_copy(table_hbm.at[idx_v.at[pl.ds(off, CF)]], rows_v, sem).wait()

        def row_body(i, carry2):
            bcol = jnp.full((16,), i, jnp.int32)
            vv0 = plsc.load_gather(vals_v, [lanes, k * RC + bcol])
            vv1 = plsc.load_gather(vals_v, [lanes_hi, k * RC + bcol])
            for j in range(F):
                v = vv0[j] if j < 16 else vv1[j - (F - 16)]
                plsc.store_scatter(
                    outc_v,
                    [jnp.full((16,), j, jnp.int32), lanes, bcol],
                    rows_v[i * F + j, :] * v,
                )
            return carry2

        lax.fori_loop(0, RC, row_body, 0)
        pltpu.sync_copy(outc_v, out_hbm.at[:, :, pl.ds(b0 + k * RC, RC)])
        return carry

    lax.fori_loop(0, NCHUNK, chunk_body, 0)


def kernel(ids, vals, table):
    tail = table[NFEAT - 64:].reshape(8, 128)
    table_rm = _table_transpose(table.T, tail).reshape(TROWS, NEMB)
    out_t = _emb_lookup(ids.astype(jnp.int32).T, vals.T, table_rm)
    return jnp.transpose(out_t, (2, 0, 1))


# speedup vs baseline: 4.9460x; 1.2468x over previous
import functools

import jax
import jax.numpy as jnp
from jax import lax
from jax.experimental import pallas as pl
from jax.experimental.pallas import tpu as pltpu
from jax.experimental.pallas import tpu_sc as plsc

NFEAT = 1000000
NEMB = 16
B = 16384
F = 26
NW = 32
RW = B // NW
RC = 64
NCHUNK = RW // RC
CF = RC * F

_mesh = plsc.VectorSubcoreMesh(core_axis_name="c", subcore_axis_name="s")

TROWS = 1000064
TFLAT = TROWS * NEMB // 128
TCOL = 31232
TCB = 512
TNCH = TCOL // TCB
TTAIL = NFEAT - NW * TCOL


TOROW = TCB * NEMB // 128


@functools.partial(
    pl.kernel,
    out_type=jax.ShapeDtypeStruct((TFLAT, 128), jnp.float32),
    mesh=_mesh,
    compiler_params=pltpu.CompilerParams(use_tc_tiling_on_sc=True, needs_layout_passes=False),
    scratch_types=[
        pltpu.VMEM((NEMB, TCB), jnp.float32),
        pltpu.VMEM((NEMB, TCB), jnp.float32),
        pltpu.VMEM((TOROW, 128), jnp.float32),
        pltpu.VMEM((TOROW, 128), jnp.float32),
        pltpu.SemaphoreType.DMA,
        pltpu.SemaphoreType.DMA,
        pltpu.SemaphoreType.DMA,
        pltpu.SemaphoreType.DMA,
    ],
)
def _table_transpose(tab_hbm, tail_hbm, out_hbm,
                     in0, in1, o0, o1, si0, si1, so0, so1):
    wid = lax.axis_index("s") * 2 + lax.axis_index("c")
    start = wid * TCOL
    lanes = lax.iota(jnp.int32, 16)
    nch = TNCH + jnp.where(wid == NW - 1, 1, 0)

    def in_slice(k):
        c0 = pl.multiple_of(start + k * TCB, 64)
        return tab_hbm.at[:, pl.ds(c0, TCB)]

    def out_slice(k):
        r0 = pl.multiple_of((start + k * TCB) * NEMB // 128, 8)
        return out_hbm.at[pl.ds(r0, TOROW), :]

    def compute(src, dst):
        def grp_body(g, carry2):
            cg = g * 16
            rows16 = (cg + lanes) >> 3
            colbase = ((cg + lanes) & 7) * 16
            for e in range(NEMB):
                plsc.store_scatter(dst, [rows16, colbase + e],
                                   src[e, pl.ds(cg, 16)])
            return carry2

        lax.fori_loop(0, TCB // 16, grp_body, 0)

    pltpu.async_copy(in_slice(0), in0, si0)

    def body(k, carry):
        @pl.when(k + 1 < nch)
        def _prefetch():
            @pl.when(k % 2 == 0)
            def _(): pltpu.async_copy(in_slice(k + 1), in1, si1)
            @pl.when(k % 2 == 1)
            def _(): pltpu.async_copy(in_slice(k + 1), in0, si0)

        def stage(inb, ob, sin, sout):
            pltpu.make_async_copy(in_slice(k), inb, sin).wait()
            @pl.when(k >= 2)
            def _(): pltpu.make_async_copy(ob, out_slice(k - 2), sout).wait()
            compute(inb, ob)
            pltpu.async_copy(ob, out_slice(k), sout)

        @pl.when(k % 2 == 0)
        def _(): stage(in0, o0, si0, so0)
        @pl.when(k % 2 == 1)
        def _(): stage(in1, o1, si1, so1)
        return carry

    lax.fori_loop(0, nch, body, 0)

    @pl.when(nch % 2 == 0)
    def _drain_even():
        pltpu.make_async_copy(o0, out_slice(nch - 2), so0).wait()
        pltpu.make_async_copy(o1, out_slice(nch - 1), so1).wait()

    @pl.when(nch % 2 == 1)
    def _drain_odd():
        pltpu.make_async_copy(o1, out_slice(nch - 2), so1).wait()
        pltpu.make_async_copy(o0, out_slice(nch - 1), so0).wait()

    @pl.when(wid == NW - 1)
    def _tail():
        pltpu.sync_copy(tail_hbm, out_hbm.at[pl.ds((NFEAT - 64) * NEMB // 128, 8), :])


@functools.partial(
    pl.kernel,
    out_type=jax.ShapeDtypeStruct((F, NEMB, B), jnp.float32),
    mesh=_mesh,
    compiler_params=pltpu.CompilerParams(use_tc_tiling_on_sc=False, needs_layout_passes=False),
    scratch_types=[
        pltpu.VMEM((F, RW), jnp.int32),
        pltpu.VMEM((F, RW), jnp.float32),
        pltpu.VMEM((RW * F,), jnp.int32),
        pltpu.VMEM((CF, NEMB), jnp.float32),
        pltpu.VMEM((F, NEMB, RC), jnp.float32),
        pltpu.SemaphoreType.DMA,
    ],
)
def _emb_lookup(ids_hbm, vals_hbm, table_hbm, out_hbm,
                ids_v, vals_v, idx_v, rows_v, outc_v, sem):
    wid = lax.axis_index("s") * 2 + lax.axis_index("c")
    b0 = wid * RW
    pltpu.sync_copy(ids_hbm.at[:, pl.ds(b0, RW)], ids_v)
    pltpu.sync_copy(vals_hbm.at[:, pl.ds(b0, RW)], vals_v)

    lanes = lax.iota(jnp.int32, 16)
    lanes_hi = lanes + (F - 16)

    def flat_body(i, carry):
        a0 = plsc.load_gather(ids_v, [lanes, jnp.full((16,), i, jnp.int32)])
        a1 = plsc.load_gather(ids_v, [lanes_hi, jnp.full((16,), i, jnp.int32)])
        plsc.store_scatter(idx_v, [i * F + lanes], a0)
        plsc.store_scatter(idx_v, [i * F + (F - 16) + lanes], a1)
        return carry

    lax.fori_loop(0, RW, flat_body, 0)

    def chunk_body(k, carry):
        off = pl.multiple_of(k * CF, 8)
        pltpu.async_copy(table_hbm.at[idx_v.at[pl.ds(off, CF)]], rows_v, sem).wait()

        def row_body(i, carry2):
            bcol = jnp.full((16,), i, jnp.int32)
            vv0 = plsc.load_gather(vals_v, [lanes, k * RC + bcol])
            vv1 = plsc.load_gather(vals_v, [lanes_hi, k * RC + bcol])
            for j in range(F):
                v = vv0[j] if j < 16 else vv1[j - (F - 16)]
                plsc.store_scatter(
                    outc_v,
                    [jnp.full((16,), j, jnp.int32), lanes, bcol],
                    rows_v[i * F + j, :] * v,
                )
            return carry2

        lax.fori_loop(0, RC, row_body, 0)
        pltpu.sync_copy(outc_v, out_hbm.at[:, :, pl.ds(b0 + k * RC, RC)])
        return carry

    lax.fori_loop(0, NCHUNK, chunk_body, 0)


def kernel(ids, vals, table):
    tail = table[NFEAT - 64:].reshape(8, 128)
    table_rm = _table_transpose(table.T, tail).reshape(TROWS, NEMB)
    out_t = _emb_lookup(ids.astype(jnp.int32).T, vals.T, table_rm)
    return jnp.transpose(out_t, (2, 0, 1))


# speedup vs baseline: 5.3082x; 1.0732x over previous
import functools

import jax
import jax.numpy as jnp
from jax import lax
from jax.experimental import pallas as pl
from jax.experimental.pallas import tpu as pltpu
from jax.experimental.pallas import tpu_sc as plsc

NFEAT = 1000000
NEMB = 16
B = 16384
F = 26
NW = 32
RW = B // NW
RC = 32
NCHUNK = RW // RC
CF = RC * F

_mesh = plsc.VectorSubcoreMesh(core_axis_name="c", subcore_axis_name="s")

TROWS = 1000064
TFLAT = TROWS * NEMB // 128
TCOL = 31232
TCB = 512
TNCH = TCOL // TCB
TTAIL = NFEAT - NW * TCOL


TOROW = TCB * NEMB // 128


@functools.partial(
    pl.kernel,
    out_type=jax.ShapeDtypeStruct((TFLAT, 128), jnp.float32),
    mesh=_mesh,
    compiler_params=pltpu.CompilerParams(use_tc_tiling_on_sc=True, needs_layout_passes=False),
    scratch_types=[
        pltpu.VMEM((NEMB, TCB), jnp.float32),
        pltpu.VMEM((NEMB, TCB), jnp.float32),
        pltpu.VMEM((TOROW, 128), jnp.float32),
        pltpu.VMEM((TOROW, 128), jnp.float32),
        pltpu.SemaphoreType.DMA,
        pltpu.SemaphoreType.DMA,
        pltpu.SemaphoreType.DMA,
        pltpu.SemaphoreType.DMA,
    ],
)
def _table_transpose(tab_hbm, tail_hbm, out_hbm,
                     in0, in1, o0, o1, si0, si1, so0, so1):
    wid = lax.axis_index("s") * 2 + lax.axis_index("c")
    start = wid * TCOL
    lanes = lax.iota(jnp.int32, 16)
    nch = TNCH + jnp.where(wid == NW - 1, 1, 0)

    def in_slice(k):
        c0 = pl.multiple_of(start + k * TCB, 64)
        return tab_hbm.at[:, pl.ds(c0, TCB)]

    def out_slice(k):
        r0 = pl.multiple_of((start + k * TCB) * NEMB // 128, 8)
        return out_hbm.at[pl.ds(r0, TOROW), :]

    def compute(src, dst):
        def grp_body(g, carry2):
            cg = g * 16
            rows16 = (cg + lanes) >> 3
            colbase = ((cg + lanes) & 7) * 16
            for e in range(NEMB):
                plsc.store_scatter(dst, [rows16, colbase + e],
                                   src[e, pl.ds(cg, 16)])
            return carry2

        lax.fori_loop(0, TCB // 16, grp_body, 0)

    pltpu.async_copy(in_slice(0), in0, si0)

    def body(k, carry):
        @pl.when(k + 1 < nch)
        def _prefetch():
            @pl.when(k % 2 == 0)
            def _(): pltpu.async_copy(in_slice(k + 1), in1, si1)
            @pl.when(k % 2 == 1)
            def _(): pltpu.async_copy(in_slice(k + 1), in0, si0)

        def stage(inb, ob, sin, sout):
            pltpu.make_async_copy(in_slice(k), inb, sin).wait()
            @pl.when(k >= 2)
            def _(): pltpu.make_async_copy(ob, out_slice(k - 2), sout).wait()
            compute(inb, ob)
            pltpu.async_copy(ob, out_slice(k), sout)

        @pl.when(k % 2 == 0)
        def _(): stage(in0, o0, si0, so0)
        @pl.when(k % 2 == 1)
        def _(): stage(in1, o1, si1, so1)
        return carry

    lax.fori_loop(0, nch, body, 0)

    @pl.when(nch % 2 == 0)
    def _drain_even():
        pltpu.make_async_copy(o0, out_slice(nch - 2), so0).wait()
        pltpu.make_async_copy(o1, out_slice(nch - 1), so1).wait()

    @pl.when(nch % 2 == 1)
    def _drain_odd():
        pltpu.make_async_copy(o1, out_slice(nch - 2), so1).wait()
        pltpu.make_async_copy(o0, out_slice(nch - 1), so0).wait()

    @pl.when(wid == NW - 1)
    def _tail():
        pltpu.sync_copy(tail_hbm, out_hbm.at[pl.ds((NFEAT - 64) * NEMB // 128, 8), :])


@functools.partial(
    pl.kernel,
    out_type=jax.ShapeDtypeStruct((F, NEMB, B), jnp.float32),
    mesh=_mesh,
    compiler_params=pltpu.CompilerParams(use_tc_tiling_on_sc=False, needs_layout_passes=False),
    scratch_types=[
        pltpu.VMEM((F, RW), jnp.int32),
        pltpu.VMEM((F, RW), jnp.float32),
        pltpu.VMEM((RW * F,), jnp.int32),
        pltpu.VMEM((CF, NEMB), jnp.float32),
        pltpu.VMEM((CF, NEMB), jnp.float32),
        pltpu.VMEM((F, NEMB, RC), jnp.float32),
        pltpu.VMEM((F, NEMB, RC), jnp.float32),
        pltpu.SemaphoreType.DMA,
        pltpu.SemaphoreType.DMA,
        pltpu.SemaphoreType.DMA,
        pltpu.SemaphoreType.DMA,
    ],
)
def _emb_lookup(ids_hbm, vals_hbm, table_hbm, out_hbm,
                ids_v, vals_v, idx_v, r0v, r1v, oc0, oc1, sg0, sg1, so0, so1):
    wid = lax.axis_index("s") * 2 + lax.axis_index("c")
    b0 = wid * RW
    pltpu.sync_copy(ids_hbm.at[:, pl.ds(b0, RW)], ids_v)
    pltpu.sync_copy(vals_hbm.at[:, pl.ds(b0, RW)], vals_v)

    lanes = lax.iota(jnp.int32, 16)
    lanes_hi = lanes + (F - 16)

    def flat_body(i, carry):
        a0 = plsc.load_gather(ids_v, [lanes, jnp.full((16,), i, jnp.int32)])
        a1 = plsc.load_gather(ids_v, [lanes_hi, jnp.full((16,), i, jnp.int32)])
        plsc.store_scatter(idx_v, [i * F + lanes], a0)
        plsc.store_scatter(idx_v, [i * F + (F - 16) + lanes], a1)
        return carry

    lax.fori_loop(0, RW, flat_body, 0)

    def gsrc(k):
        off = pl.multiple_of(k * CF, 8)
        return table_hbm.at[idx_v.at[pl.ds(off, CF)]]

    def osl(k):
        return out_hbm.at[:, :, pl.ds(b0 + k * RC, RC)]

    def compute(k, rows_v, outc_v):
        def row_body(i, carry2):
            bcol = jnp.full((16,), i, jnp.int32)
            vv0 = plsc.load_gather(vals_v, [lanes, k * RC + bcol])
            vv1 = plsc.load_gather(vals_v, [lanes_hi, k * RC + bcol])
            for j in range(F):
                v = vv0[j] if j < 16 else vv1[j - (F - 16)]
                plsc.store_scatter(
                    outc_v,
                    [jnp.full((16,), j, jnp.int32), lanes, bcol],
                    rows_v[i * F + j, :] * v,
                )
            return carry2

        lax.fori_loop(0, RC, row_body, 0)

    pltpu.async_copy(gsrc(0), r0v, sg0)

    def chunk_body(k, carry):
        @pl.when(k + 1 < NCHUNK)
        def _prefetch():
            @pl.when(k % 2 == 0)
            def _(): pltpu.async_copy(gsrc(k + 1), r1v, sg1)
            @pl.when(k % 2 == 1)
            def _(): pltpu.async_copy(gsrc(k + 1), r0v, sg0)

        def stage(rows_v, outc_v, sg, so):
            pltpu.make_async_copy(gsrc(k), rows_v, sg).wait()
            @pl.when(k >= 2)
            def _(): pltpu.make_async_copy(outc_v, osl(k - 2), so).wait()
            compute(k, rows_v, outc_v)
            pltpu.async_copy(outc_v, osl(k), so)

        @pl.when(k % 2 == 0)
        def _(): stage(r0v, oc0, sg0, so0)
        @pl.when(k % 2 == 1)
        def _(): stage(r1v, oc1, sg1, so1)
        return carry

    lax.fori_loop(0, NCHUNK, chunk_body, 0)

    pltpu.make_async_copy(oc0, osl(NCHUNK - 2), so0).wait()
    pltpu.make_async_copy(oc1, osl(NCHUNK - 1), so1).wait()


def kernel(ids, vals, table):
    tail = table[NFEAT - 64:].reshape(8, 128)
    table_rm = _table_transpose(table.T, tail).reshape(TROWS, NEMB)
    out_t = _emb_lookup(ids.astype(jnp.int32).T, vals.T, table_rm)
    return jnp.transpose(out_t, (2, 0, 1))
